# baseline (speedup 1.0000x reference)
import functools

import jax
import jax.numpy as jnp
from jax import lax
from jax.experimental import pallas as pl
from jax.experimental.pallas import tpu as pltpu
from jax.experimental.pallas import tpu_sc as plsc

B, H, W, C = 16, 384, 384, 3
N = B * H * W
NTILES = 32
PER_TILE = N // NTILES
SUB = 8192
NSUB = PER_TILE // SUB


def _sampler_body(coords_hbm, imgs_hbm, out_hbm, cbuf,
                  ib00, ib01, ib10, ib11, gb00, gb01, gb10, gb11,
                  fbx, fby, obuf, sem):
    wid = lax.axis_index("s") * 2 + lax.axis_index("c")
    ebase = wid * PER_TILE
    bflat = (wid // 2) * (H * W)

    def subchunk(s, carry):
        e0 = pl.multiple_of(ebase + s * SUB, SUB)
        pltpu.sync_copy(coords_hbm.at[pl.ds(e0 * 2, 2 * SUB)], cbuf)

        def compute(i, carry2):
            iota = lax.iota(jnp.int32, 16)
            xsel = i * 32 + iota * 2
            xv = plsc.load_gather(cbuf, [xsel])
            yv = plsc.load_gather(cbuf, [xsel + 1])
            x0 = xv.astype(jnp.int32)
            y0 = yv.astype(jnp.int32)
            fx = xv - x0.astype(jnp.float32)
            fy = yv - y0.astype(jnp.float32)
            x0c = jnp.minimum(x0, W - 1)
            x1c = jnp.minimum(x0 + 1, W - 1)
            r0 = bflat + jnp.minimum(y0, H - 1) * W
            r1 = bflat + jnp.minimum(y0 + 1, H - 1) * W
            c = pl.multiple_of(i * 16, 16)
            ib00[pl.ds(c, 16)] = r0 + x0c
            ib01[pl.ds(c, 16)] = r1 + x0c
            ib10[pl.ds(c, 16)] = r0 + x1c
            ib11[pl.ds(c, 16)] = r1 + x1c
            fbx[pl.ds(c, 16)] = fx
            fby[pl.ds(c, 16)] = fy
            return carry2

        lax.fori_loop(0, SUB // 16, compute, 0)

        pltpu.async_copy(imgs_hbm.at[ib00], gb00, sem)
        pltpu.async_copy(imgs_hbm.at[ib01], gb01, sem)
        pltpu.async_copy(imgs_hbm.at[ib10], gb10, sem)
        pltpu.async_copy(imgs_hbm.at[ib11], gb11, sem)
        pltpu.make_async_copy(imgs_hbm.at[ib00], gb00, sem).wait()
        pltpu.make_async_copy(imgs_hbm.at[ib01], gb01, sem).wait()
        pltpu.make_async_copy(imgs_hbm.at[ib10], gb10, sem).wait()
        pltpu.make_async_copy(imgs_hbm.at[ib11], gb11, sem).wait()

        def combine(i, carry2):
            c = pl.multiple_of(i * 16, 16)
            g00 = gb00[pl.ds(c, 16)]
            g01 = gb01[pl.ds(c, 16)]
            g10 = gb10[pl.ds(c, 16)]
            g11 = gb11[pl.ds(c, 16)]
            fx = fbx[pl.ds(c, 16)]
            fy = fby[pl.ds(c, 16)]
            wx0 = 1.0 - fx
            wy0 = 1.0 - fy
            res = (wx0 * wy0) * g00 + (wx0 * fy) * g01
            res = res + ((fx * wy0) * g10 + (fx * fy) * g11)
            obuf[pl.ds(c, 16)] = res
            return carry2

        lax.fori_loop(0, SUB // 16, combine, 0)
        pltpu.sync_copy(obuf, out_hbm.at[pl.ds(e0, SUB)])
        return carry

    lax.fori_loop(0, NSUB, subchunk, 0)


def kernel(imgs, coords):
    flat = imgs.reshape(-1)
    cflat = coords.reshape(-1)
    mesh = plsc.VectorSubcoreMesh(core_axis_name="c", subcore_axis_name="s")
    run = functools.partial(
        pl.kernel,
        mesh=mesh,
        compiler_params=pltpu.CompilerParams(needs_layout_passes=False),
        out_type=jax.ShapeDtypeStruct((N,), jnp.float32),
        scratch_types=[
            pltpu.VMEM((2 * SUB,), jnp.float32),
            pltpu.VMEM((SUB,), jnp.int32),
            pltpu.VMEM((SUB,), jnp.int32),
            pltpu.VMEM((SUB,), jnp.int32),
            pltpu.VMEM((SUB,), jnp.int32),
            pltpu.VMEM((SUB,), jnp.float32),
            pltpu.VMEM((SUB,), jnp.float32),
            pltpu.VMEM((SUB,), jnp.float32),
            pltpu.VMEM((SUB,), jnp.float32),
            pltpu.VMEM((SUB,), jnp.float32),
            pltpu.VMEM((SUB,), jnp.float32),
            pltpu.VMEM((SUB,), jnp.float32),
            pltpu.SemaphoreType.DMA,
        ],
    )(_sampler_body)
    out = run(cflat, flat)
    return out.reshape(B, H, W, 1)


# speedup vs baseline: 1.4707x; 1.4707x over previous
import functools

import jax
import jax.numpy as jnp
from jax import lax
from jax.experimental import pallas as pl
from jax.experimental.pallas import tpu as pltpu
from jax.experimental.pallas import tpu_sc as plsc

B, H, W, C = 16, 384, 384, 3
N = B * H * W
NTILES = 32
PER_TILE = N // NTILES
SUB = 8192
NSUB = PER_TILE // SUB
SC_WORDS = 8 * H * W // 2
HIMASK = jnp.int32(-65536)


def _sampler_body(coords_hbm, src_hbm, out_hbm, cbuf,
                  ib00, ib01, ib10, ib11, gb00, gb01, gb10, gb11,
                  obuf, shared, sem):
    cix = lax.axis_index("c")
    six = lax.axis_index("s")
    wid = cix * 16 + six
    ebase = wid * PER_TILE
    bflat = (six // 2) * (H * W)

    strip = SC_WORDS // 16
    pltpu.async_copy(
        src_hbm.at[pl.ds(cix * SC_WORDS + six * strip, strip)],
        shared.at[pl.ds(six * strip, strip)], sem).wait()
    plsc.subcore_barrier()

    def subchunk(s, carry):
        e0 = pl.multiple_of(ebase + s * SUB, SUB)
        pltpu.sync_copy(coords_hbm.at[pl.ds(e0 * 2, 2 * SUB)], cbuf)

        def compute(i, carry2):
            iota = lax.iota(jnp.int32, 16)
            xsel = i * 32 + iota * 2
            xv = plsc.load_gather(cbuf, [xsel])
            yv = plsc.load_gather(cbuf, [xsel + 1])
            x0 = xv.astype(jnp.int32)
            y0 = yv.astype(jnp.int32)
            x0c = jnp.minimum(x0, W - 1)
            x1c = jnp.minimum(x0 + 1, W - 1)
            r0 = bflat + jnp.minimum(y0, H - 1) * W
            r1 = bflat + jnp.minimum(y0 + 1, H - 1) * W
            c = pl.multiple_of(i * 16, 16)
            ib00[pl.ds(c, 16)] = (r0 + x0c) >> 1
            ib01[pl.ds(c, 16)] = (r1 + x0c) >> 1
            ib10[pl.ds(c, 16)] = (r0 + x1c) >> 1
            ib11[pl.ds(c, 16)] = (r1 + x1c) >> 1
            return carry2

        lax.fori_loop(0, SUB // 16, compute, 0)

        pltpu.async_copy(shared.at[ib00], gb00, sem)
        pltpu.async_copy(shared.at[ib01], gb01, sem)
        pltpu.async_copy(shared.at[ib10], gb10, sem)
        pltpu.async_copy(shared.at[ib11], gb11, sem)
        pltpu.make_async_copy(shared.at[ib00], gb00, sem).wait()
        pltpu.make_async_copy(shared.at[ib01], gb01, sem).wait()
        pltpu.make_async_copy(shared.at[ib10], gb10, sem).wait()
        pltpu.make_async_copy(shared.at[ib11], gb11, sem).wait()

        def combine(i, carry2):
            iota = lax.iota(jnp.int32, 16)
            c = pl.multiple_of(i * 16, 16)
            w00 = gb00[pl.ds(c, 16)]
            w01 = gb01[pl.ds(c, 16)]
            w10 = gb10[pl.ds(c, 16)]
            w11 = gb11[pl.ds(c, 16)]
            xsel = i * 32 + iota * 2
            xv = plsc.load_gather(cbuf, [xsel])
            yv = plsc.load_gather(cbuf, [xsel + 1])
            x0 = xv.astype(jnp.int32)
            y0 = yv.astype(jnp.int32)
            fx = xv - x0.astype(jnp.float32)
            fy = yv - y0.astype(jnp.float32)
            p0 = (jnp.minimum(x0, W - 1) & 1) == 1
            p1 = (jnp.minimum(x0 + 1, W - 1) & 1) == 1
            v00 = plsc.bitcast(jnp.where(p0, w00 & HIMASK, w00 << 16), jnp.float32)
            v01 = plsc.bitcast(jnp.where(p0, w01 & HIMASK, w01 << 16), jnp.float32)
            v10 = plsc.bitcast(jnp.where(p1, w10 & HIMASK, w10 << 16), jnp.float32)
            v11 = plsc.bitcast(jnp.where(p1, w11 & HIMASK, w11 << 16), jnp.float32)
            wx0 = 1.0 - fx
            wy0 = 1.0 - fy
            res = (wx0 * wy0) * v00 + (wx0 * fy) * v01
            res = res + ((fx * wy0) * v10 + (fx * fy) * v11)
            obuf[pl.ds(c, 16)] = res
            return carry2

        lax.fori_loop(0, SUB // 16, combine, 0)
        pltpu.sync_copy(obuf, out_hbm.at[pl.ds(e0, SUB)])
        return carry

    lax.fori_loop(0, NSUB, subchunk, 0)


def kernel(imgs, coords):
    src16 = imgs.reshape(-1)[:N].astype(jnp.bfloat16)
    srcw = lax.bitcast_convert_type(src16.reshape(N // 2, 2), jnp.int32)
    cflat = coords.reshape(-1)
    mesh = plsc.VectorSubcoreMesh(core_axis_name="c", subcore_axis_name="s")
    run = functools.partial(
        pl.kernel,
        mesh=mesh,
        compiler_params=pltpu.CompilerParams(needs_layout_passes=False),
        out_type=jax.ShapeDtypeStruct((N,), jnp.float32),
        scratch_types=[
            pltpu.VMEM((2 * SUB,), jnp.float32),
            pltpu.VMEM((SUB,), jnp.int32),
            pltpu.VMEM((SUB,), jnp.int32),
            pltpu.VMEM((SUB,), jnp.int32),
            pltpu.VMEM((SUB,), jnp.int32),
            pltpu.VMEM((SUB,), jnp.int32),
            pltpu.VMEM((SUB,), jnp.int32),
            pltpu.VMEM((SUB,), jnp.int32),
            pltpu.VMEM((SUB,), jnp.int32),
            pltpu.VMEM((SUB,), jnp.float32),
            pltpu.VMEM_SHARED((SC_WORDS,), jnp.int32),
            pltpu.SemaphoreType.DMA,
        ],
    )(_sampler_body)
    out = run(cflat, srcw)
    return out.reshape(B, H, W, 1)


# speedup vs baseline: 1.4823x; 1.0079x over previous
import functools

import jax
import jax.numpy as jnp
from jax import lax
from jax.experimental import pallas as pl
from jax.experimental.pallas import tpu as pltpu
from jax.experimental.pallas import tpu_sc as plsc

B, H, W, C = 16, 384, 384, 3
N = B * H * W
NTILES = 32
PER_TILE = N // NTILES
SUB = 8192
NSUB = PER_TILE // SUB
STEP = 128
NSTEP = SUB // STEP
SC_WORDS = 8 * H * W // 2
HIMASK = jnp.int32(-65536)


def _sampler_body(coords_hbm, src_hbm, out_hbm, cbuf,
                  ib00, ib01, ib10, ib11, gb00, gb01, gb10, gb11,
                  obuf, shared, sem):
    cix = lax.axis_index("c")
    six = lax.axis_index("s")
    wid = cix * 16 + six
    ebase = wid * PER_TILE
    bflat = (six // 2) * (H * W)

    strip = SC_WORDS // 16
    pltpu.async_copy(
        src_hbm.at[pl.ds(cix * SC_WORDS + six * strip, strip)],
        shared.at[pl.ds(six * strip, strip)], sem).wait()
    plsc.subcore_barrier()

    def subchunk(s, carry):
        e0 = pl.multiple_of(ebase + s * SUB, SUB)
        pltpu.sync_copy(coords_hbm.at[pl.ds(e0 * 2, 2 * SUB)], cbuf)

        def compute(j, carry2):
            iota = lax.iota(jnp.int32, 16)
            for i2 in range(STEP // 16):
                xsel = j * (2 * STEP) + i2 * 32 + iota * 2
                xv = plsc.load_gather(cbuf, [xsel])
                yv = plsc.load_gather(cbuf, [xsel + 1])
                x0 = xv.astype(jnp.int32)
                y0 = yv.astype(jnp.int32)
                x0c = jnp.minimum(x0, W - 1)
                x1c = jnp.minimum(x0 + 1, W - 1)
                r0 = bflat + jnp.minimum(y0, H - 1) * W
                r1 = bflat + jnp.minimum(y0 + 1, H - 1) * W
                c = i2 * 16
                ib00[j, pl.ds(c, 16)] = (r0 + x0c) >> 1
                ib01[j, pl.ds(c, 16)] = (r1 + x0c) >> 1
                ib10[j, pl.ds(c, 16)] = (r0 + x1c) >> 1
                ib11[j, pl.ds(c, 16)] = (r1 + x1c) >> 1
            pltpu.async_copy(shared.at[ib00.at[j]], gb00.at[j], sem)
            pltpu.async_copy(shared.at[ib01.at[j]], gb01.at[j], sem)
            pltpu.async_copy(shared.at[ib10.at[j]], gb10.at[j], sem)
            pltpu.async_copy(shared.at[ib11.at[j]], gb11.at[j], sem)
            return carry2

        lax.fori_loop(0, NSTEP, compute, 0)

        def drain(j, carry2):
            pltpu.make_async_copy(shared.at[ib00.at[j]], gb00.at[j], sem).wait()
            pltpu.make_async_copy(shared.at[ib01.at[j]], gb01.at[j], sem).wait()
            pltpu.make_async_copy(shared.at[ib10.at[j]], gb10.at[j], sem).wait()
            pltpu.make_async_copy(shared.at[ib11.at[j]], gb11.at[j], sem).wait()
            return carry2

        lax.fori_loop(0, NSTEP, drain, 0)

        def combine(j, carry2):
            iota = lax.iota(jnp.int32, 16)
            for i2 in range(STEP // 16):
                c = i2 * 16
                w00 = gb00[j, pl.ds(c, 16)]
                w01 = gb01[j, pl.ds(c, 16)]
                w10 = gb10[j, pl.ds(c, 16)]
                w11 = gb11[j, pl.ds(c, 16)]
                xsel = j * (2 * STEP) + i2 * 32 + iota * 2
                xv = plsc.load_gather(cbuf, [xsel])
                yv = plsc.load_gather(cbuf, [xsel + 1])
                x0 = xv.astype(jnp.int32)
                y0 = yv.astype(jnp.int32)
                fx = xv - x0.astype(jnp.float32)
                fy = yv - y0.astype(jnp.float32)
                p0 = (jnp.minimum(x0, W - 1) & 1) == 1
                p1 = (jnp.minimum(x0 + 1, W - 1) & 1) == 1
                v00 = plsc.bitcast(jnp.where(p0, w00 & HIMASK, w00 << 16), jnp.float32)
                v01 = plsc.bitcast(jnp.where(p0, w01 & HIMASK, w01 << 16), jnp.float32)
                v10 = plsc.bitcast(jnp.where(p1, w10 & HIMASK, w10 << 16), jnp.float32)
                v11 = plsc.bitcast(jnp.where(p1, w11 & HIMASK, w11 << 16), jnp.float32)
                wx0 = 1.0 - fx
                wy0 = 1.0 - fy
                res = (wx0 * wy0) * v00 + (wx0 * fy) * v01
                res = res + ((fx * wy0) * v10 + (fx * fy) * v11)
                obuf[pl.ds(j * STEP + c, 16)] = res
            return carry2

        lax.fori_loop(0, NSTEP, combine, 0)
        pltpu.sync_copy(obuf, out_hbm.at[pl.ds(e0, SUB)])
        return carry

    lax.fori_loop(0, NSUB, subchunk, 0)


def kernel(imgs, coords):
    src16 = imgs.reshape(-1)[:N].astype(jnp.bfloat16)
    srcw = lax.bitcast_convert_type(src16.reshape(N // 2, 2), jnp.int32)
    cflat = coords.reshape(-1)
    mesh = plsc.VectorSubcoreMesh(core_axis_name="c", subcore_axis_name="s")
    run = functools.partial(
        pl.kernel,
        mesh=mesh,
        compiler_params=pltpu.CompilerParams(needs_layout_passes=False),
        out_type=jax.ShapeDtypeStruct((N,), jnp.float32),
        scratch_types=[
            pltpu.VMEM((2 * SUB,), jnp.float32),
            pltpu.VMEM((NSTEP, STEP), jnp.int32),
            pltpu.VMEM((NSTEP, STEP), jnp.int32),
            pltpu.VMEM((NSTEP, STEP), jnp.int32),
            pltpu.VMEM((NSTEP, STEP), jnp.int32),
            pltpu.VMEM((NSTEP, STEP), jnp.int32),
            pltpu.VMEM((NSTEP, STEP), jnp.int32),
            pltpu.VMEM((NSTEP, STEP), jnp.int32),
            pltpu.VMEM((NSTEP, STEP), jnp.int32),
            pltpu.VMEM((SUB,), jnp.float32),
            pltpu.VMEM_SHARED((SC_WORDS,), jnp.int32),
            pltpu.SemaphoreType.DMA,
        ],
    )(_sampler_body)
    out = run(cflat, srcw)
    return out.reshape(B, H, W, 1)


# speedup vs baseline: 1.5192x; 1.0249x over previous
import functools

import jax
import jax.numpy as jnp
from jax import lax
from jax.experimental import pallas as pl
from jax.experimental.pallas import tpu as pltpu
from jax.experimental.pallas import tpu_sc as plsc

B, H, W, C = 16, 384, 384, 3
N = B * H * W
NTILES = 32
PER_TILE = N // NTILES
SUB = 3072
NSUB = PER_TILE // SUB
RPR = W // 12
ROWS_B = H * RPR
SC_ROWS = 8 * ROWS_B
HIMASK = jnp.int32(-65536)


def _sampler_body(coords_hbm, tab_hbm, out_hbm, cbuf,
                  ib0, ib1, gr0, gr1, obuf, shared, sem):
    cix = lax.axis_index("c")
    six = lax.axis_index("s")
    wid = cix * 16 + six
    ebase = wid * PER_TILE
    rowbase = (six // 2) * ROWS_B

    strip = SC_ROWS // 16
    pltpu.async_copy(
        tab_hbm.at[pl.ds(cix * SC_ROWS + six * strip, strip)],
        shared.at[pl.ds(six * strip, strip)], sem).wait()
    plsc.subcore_barrier()

    def subchunk(s, carry):
        e0 = pl.multiple_of(ebase + s * SUB, SUB)
        pltpu.sync_copy(coords_hbm.at[pl.ds(e0 * 2, 2 * SUB)], cbuf)

        def compute(i, carry2):
            iota = lax.iota(jnp.int32, 16)
            xsel = i * 32 + iota * 2
            xv = plsc.load_gather(cbuf, [xsel])
            yv = plsc.load_gather(cbuf, [xsel + 1])
            x0 = xv.astype(jnp.int32)
            y0 = yv.astype(jnp.int32)
            x0c = jnp.minimum(x0, W - 1)
            xd12 = (x0c * 683) >> 13
            y0c = jnp.minimum(y0, H - 1)
            y1c = jnp.minimum(y0 + 1, H - 1)
            c = pl.multiple_of(i * 16, 16)
            ib0[pl.ds(c, 16)] = rowbase + y0c * RPR + xd12
            ib1[pl.ds(c, 16)] = rowbase + y1c * RPR + xd12
            return carry2

        lax.fori_loop(0, SUB // 16, compute, 0)

        pltpu.async_copy(shared.at[ib0], gr0, sem)
        pltpu.async_copy(shared.at[ib1], gr1, sem)
        pltpu.make_async_copy(shared.at[ib0], gr0, sem).wait()
        pltpu.make_async_copy(shared.at[ib1], gr1, sem).wait()

        def combine(i, carry2):
            iota = lax.iota(jnp.int32, 16)
            c = pl.multiple_of(i * 16, 16)
            ridx = c + iota
            xsel = i * 32 + iota * 2
            xv = plsc.load_gather(cbuf, [xsel])
            yv = plsc.load_gather(cbuf, [xsel + 1])
            x0 = xv.astype(jnp.int32)
            y0 = yv.astype(jnp.int32)
            fx = xv - x0.astype(jnp.float32)
            fy = yv - y0.astype(jnp.float32)
            x0c = jnp.minimum(x0, W - 1)
            xd12 = (x0c * 683) >> 13
            o = x0c - xd12 * 12
            wl = o >> 1
            wr = (o + 1) >> 1
            a0 = plsc.load_gather(gr0, [ridx, wl])
            b0 = plsc.load_gather(gr0, [ridx, wr])
            a1 = plsc.load_gather(gr1, [ridx, wl])
            b1 = plsc.load_gather(gr1, [ridx, wr])
            pL = (o & 1) == 1
            v00 = plsc.bitcast(jnp.where(pL, a0 & HIMASK, a0 << 16), jnp.float32)
            v10 = plsc.bitcast(jnp.where(pL, b0 << 16, b0 & HIMASK), jnp.float32)
            v01 = plsc.bitcast(jnp.where(pL, a1 & HIMASK, a1 << 16), jnp.float32)
            v11 = plsc.bitcast(jnp.where(pL, b1 << 16, b1 & HIMASK), jnp.float32)
            wx0 = 1.0 - fx
            wy0 = 1.0 - fy
            res = (wx0 * wy0) * v00 + (wx0 * fy) * v01
            res = res + ((fx * wy0) * v10 + (fx * fy) * v11)
            obuf[pl.ds(c, 16)] = res
            return carry2

        lax.fori_loop(0, SUB // 16, combine, 0)
        pltpu.sync_copy(obuf, out_hbm.at[pl.ds(e0, SUB)])
        return carry

    lax.fori_loop(0, NSUB, subchunk, 0)


def kernel(imgs, coords):
    src16 = imgs.reshape(-1)[:N].astype(jnp.bfloat16)
    rows = N // 12
    vpad = jnp.concatenate([src16, jnp.zeros((16,), jnp.bfloat16)])
    tab16 = jnp.stack([vpad[j:j + 12 * rows:12] for j in range(16)], axis=1)
    tabw = lax.bitcast_convert_type(tab16.reshape(rows, 8, 2), jnp.int32)
    cflat = coords.reshape(-1)
    mesh = plsc.VectorSubcoreMesh(core_axis_name="c", subcore_axis_name="s")
    run = functools.partial(
        pl.kernel,
        mesh=mesh,
        compiler_params=pltpu.CompilerParams(
            needs_layout_passes=False, use_tc_tiling_on_sc=False),
        out_type=jax.ShapeDtypeStruct((N,), jnp.float32),
        scratch_types=[
            pltpu.VMEM((2 * SUB,), jnp.float32),
            pltpu.VMEM((SUB,), jnp.int32),
            pltpu.VMEM((SUB,), jnp.int32),
            pltpu.VMEM((SUB, 8), jnp.int32),
            pltpu.VMEM((SUB, 8), jnp.int32),
            pltpu.VMEM((SUB,), jnp.float32),
            pltpu.VMEM_SHARED((SC_ROWS, 8), jnp.int32),
            pltpu.SemaphoreType.DMA,
        ],
    )(_sampler_body)
    out = run(cflat, tabw)
    return out.reshape(B, H, W, 1)


# speedup vs baseline: 2.0188x; 1.3289x over previous
import functools

import jax
import jax.numpy as jnp
from jax import lax
from jax.experimental import pallas as pl
from jax.experimental.pallas import tpu as pltpu
from jax.experimental.pallas import tpu_sc as plsc

B, H, W, C = 16, 384, 384, 3
N = B * H * W
NTILES = 32
PER_TILE = N // NTILES
SUB = 3072
NSUB = PER_TILE // SUB
UNROLL = 4
RPR = W // 12
ROWS_B = H * RPR
SC_ROWS = 8 * ROWS_B
HIMASK = jnp.int32(-65536)


def _sampler_body(cx_hbm, cy_hbm, tab_hbm, out_hbm, cbx, cby,
                  ib0, ib1, gr0, gr1, obuf, shared, sem):
    cix = lax.axis_index("c")
    six = lax.axis_index("s")
    wid = cix * 16 + six
    ebase = wid * PER_TILE
    rowbase = (six // 2) * ROWS_B
    iota = lax.iota(jnp.int32, 16)

    strip = SC_ROWS // 16
    pltpu.async_copy(
        tab_hbm.at[pl.ds(cix * SC_ROWS + six * strip, strip)],
        shared.at[pl.ds(six * strip, strip)], sem).wait()
    plsc.subcore_barrier()

    def subchunk(s, carry):
        e0 = pl.multiple_of(ebase + s * SUB, SUB)
        pltpu.sync_copy(cx_hbm.at[pl.ds(e0, SUB)], cbx)
        pltpu.sync_copy(cy_hbm.at[pl.ds(e0, SUB)], cby)

        def compute(i, carry2):
            for u in range(UNROLL):
                c = pl.multiple_of(i * 16 * UNROLL + u * 16, 16)
                xv = cbx[pl.ds(c, 16)]
                yv = cby[pl.ds(c, 16)]
                x0 = xv.astype(jnp.int32)
                y0 = yv.astype(jnp.int32)
                x0c = jnp.minimum(x0, W - 1)
                xd12 = (x0c * 683) >> 13
                y0c = jnp.minimum(y0, H - 1)
                y1c = jnp.minimum(y0 + 1, H - 1)
                ib0[pl.ds(c, 16)] = rowbase + y0c * RPR + xd12
                ib1[pl.ds(c, 16)] = rowbase + y1c * RPR + xd12
            return carry2

        lax.fori_loop(0, SUB // (16 * UNROLL), compute, 0)

        pltpu.async_copy(shared.at[ib0], gr0, sem)
        pltpu.async_copy(shared.at[ib1], gr1, sem)
        pltpu.make_async_copy(shared.at[ib0], gr0, sem).wait()
        pltpu.make_async_copy(shared.at[ib1], gr1, sem).wait()

        def combine(i, carry2):
            for u in range(UNROLL):
                c = pl.multiple_of(i * 16 * UNROLL + u * 16, 16)
                ridx = c + iota
                xv = cbx[pl.ds(c, 16)]
                yv = cby[pl.ds(c, 16)]
                x0 = xv.astype(jnp.int32)
                y0 = yv.astype(jnp.int32)
                fx = xv - x0.astype(jnp.float32)
                fy = yv - y0.astype(jnp.float32)
                x0c = jnp.minimum(x0, W - 1)
                xd12 = (x0c * 683) >> 13
                o = x0c - xd12 * 12
                wl = o >> 1
                wr = (o + 1) >> 1
                a0 = plsc.load_gather(gr0, [ridx, wl])
                b0 = plsc.load_gather(gr0, [ridx, wr])
                a1 = plsc.load_gather(gr1, [ridx, wl])
                b1 = plsc.load_gather(gr1, [ridx, wr])
                pL = (o & 1) == 1
                v00 = plsc.bitcast(jnp.where(pL, a0 & HIMASK, a0 << 16), jnp.float32)
                v10 = plsc.bitcast(jnp.where(pL, b0 << 16, b0 & HIMASK), jnp.float32)
                v01 = plsc.bitcast(jnp.where(pL, a1 & HIMASK, a1 << 16), jnp.float32)
                v11 = plsc.bitcast(jnp.where(pL, b1 << 16, b1 & HIMASK), jnp.float32)
                wx0 = 1.0 - fx
                wy0 = 1.0 - fy
                res = (wx0 * wy0) * v00 + (wx0 * fy) * v01
                res = res + ((fx * wy0) * v10 + (fx * fy) * v11)
                obuf[pl.ds(c, 16)] = res
            return carry2

        lax.fori_loop(0, SUB // (16 * UNROLL), combine, 0)
        pltpu.sync_copy(obuf, out_hbm.at[pl.ds(e0, SUB)])
        return carry

    lax.fori_loop(0, NSUB, subchunk, 0)


def kernel(imgs, coords):
    src16 = imgs.reshape(-1)[:N].astype(jnp.bfloat16)
    rows = N // 12
    vpad = jnp.concatenate([src16, jnp.zeros((16,), jnp.bfloat16)])
    tab16 = jnp.stack([vpad[j:j + 12 * rows:12] for j in range(16)], axis=1)
    tabw = lax.bitcast_convert_type(tab16.reshape(rows, 8, 2), jnp.int32)
    cpl = coords.reshape(N, 2)
    cx = cpl[:, 0]
    cy = cpl[:, 1]
    mesh = plsc.VectorSubcoreMesh(core_axis_name="c", subcore_axis_name="s")
    run = functools.partial(
        pl.kernel,
        mesh=mesh,
        compiler_params=pltpu.CompilerParams(
            needs_layout_passes=False, use_tc_tiling_on_sc=False),
        out_type=jax.ShapeDtypeStruct((N,), jnp.float32),
        scratch_types=[
            pltpu.VMEM((SUB,), jnp.float32),
            pltpu.VMEM((SUB,), jnp.float32),
            pltpu.VMEM((SUB,), jnp.int32),
            pltpu.VMEM((SUB,), jnp.int32),
            pltpu.VMEM((SUB, 8), jnp.int32),
            pltpu.VMEM((SUB, 8), jnp.int32),
            pltpu.VMEM((SUB,), jnp.float32),
            pltpu.VMEM_SHARED((SC_ROWS, 8), jnp.int32),
            pltpu.SemaphoreType.DMA,
        ],
    )(_sampler_body)
    out = run(cx, cy, tabw)
    return out.reshape(B, H, W, 1)


# speedup vs baseline: 2.0462x; 1.0136x over previous
import functools

import jax
import jax.numpy as jnp
from jax import lax
from jax.experimental import pallas as pl
from jax.experimental.pallas import tpu as pltpu
from jax.experimental.pallas import tpu_sc as plsc

B, H, W, C = 16, 384, 384, 3
N = B * H * W
NTILES = 32
PER_TILE = N // NTILES
SUB = 3072
NSUB = PER_TILE // SUB
UNROLL = 4
RPR = W // 12
ROWS_B = H * RPR
SC_ROWS = 8 * ROWS_B
HIMASK = jnp.int32(-65536)


def _sampler_body(cx_hbm, cy_hbm, tab_hbm, out_hbm, cbx, cby,
                  ib0, ib1, gr0, gr1, obuf, shared, sem):
    cix = lax.axis_index("c")
    six = lax.axis_index("s")
    wid = cix * 16 + six
    ebase = wid * PER_TILE
    rowbase = (six // 2) * ROWS_B
    iota = lax.iota(jnp.int32, 16)

    strip = SC_ROWS // 16
    pltpu.async_copy(
        tab_hbm.at[pl.ds(cix * SC_ROWS + six * strip, strip)],
        shared.at[pl.ds(six * strip, strip)], sem).wait()
    plsc.subcore_barrier()

    def subchunk(s, carry):
        e0 = pl.multiple_of(ebase + s * SUB, SUB)
        pltpu.sync_copy(cx_hbm.at[pl.ds(e0, SUB)], cbx)
        pltpu.sync_copy(cy_hbm.at[pl.ds(e0, SUB)], cby)

        @plsc.parallel_loop(0, SUB, step=16, unroll=UNROLL)
        def compute(i):
            if True:
                c = pl.multiple_of(i, 16)
                xv = cbx[pl.ds(c, 16)]
                yv = cby[pl.ds(c, 16)]
                x0 = xv.astype(jnp.int32)
                y0 = yv.astype(jnp.int32)
                x0c = jnp.minimum(x0, W - 1)
                xd12 = (x0c * 683) >> 13
                y0c = jnp.minimum(y0, H - 1)
                y1c = jnp.minimum(y0 + 1, H - 1)
                ib0[pl.ds(c, 16)] = rowbase + y0c * RPR + xd12
                ib1[pl.ds(c, 16)] = rowbase + y1c * RPR + xd12

        pltpu.async_copy(shared.at[ib0], gr0, sem)
        pltpu.async_copy(shared.at[ib1], gr1, sem)
        pltpu.make_async_copy(shared.at[ib0], gr0, sem).wait()
        pltpu.make_async_copy(shared.at[ib1], gr1, sem).wait()

        @plsc.parallel_loop(0, SUB, step=16, unroll=UNROLL)
        def combine(i):
            if True:
                c = pl.multiple_of(i, 16)
                ridx = c + iota
                xv = cbx[pl.ds(c, 16)]
                yv = cby[pl.ds(c, 16)]
                x0 = xv.astype(jnp.int32)
                y0 = yv.astype(jnp.int32)
                fx = xv - x0.astype(jnp.float32)
                fy = yv - y0.astype(jnp.float32)
                x0c = jnp.minimum(x0, W - 1)
                xd12 = (x0c * 683) >> 13
                o = x0c - xd12 * 12
                wl = o >> 1
                wr = (o + 1) >> 1
                a0 = plsc.load_gather(gr0, [ridx, wl])
                b0 = plsc.load_gather(gr0, [ridx, wr])
                a1 = plsc.load_gather(gr1, [ridx, wl])
                b1 = plsc.load_gather(gr1, [ridx, wr])
                pL = (o & 1) == 1
                v00 = plsc.bitcast(jnp.where(pL, a0 & HIMASK, a0 << 16), jnp.float32)
                v10 = plsc.bitcast(jnp.where(pL, b0 << 16, b0 & HIMASK), jnp.float32)
                v01 = plsc.bitcast(jnp.where(pL, a1 & HIMASK, a1 << 16), jnp.float32)
                v11 = plsc.bitcast(jnp.where(pL, b1 << 16, b1 & HIMASK), jnp.float32)
                wx0 = 1.0 - fx
                wy0 = 1.0 - fy
                res = (wx0 * wy0) * v00 + (wx0 * fy) * v01
                res = res + ((fx * wy0) * v10 + (fx * fy) * v11)
                obuf[pl.ds(c, 16)] = res
        pltpu.sync_copy(obuf, out_hbm.at[pl.ds(e0, SUB)])
        return carry

    lax.fori_loop(0, NSUB, subchunk, 0)


def kernel(imgs, coords):
    src16 = imgs.reshape(-1)[:N].astype(jnp.bfloat16)
    rows = N // 12
    vpad = jnp.concatenate([src16, jnp.zeros((16,), jnp.bfloat16)])
    tab16 = jnp.stack([vpad[j:j + 12 * rows:12] for j in range(16)], axis=1)
    tabw = lax.bitcast_convert_type(tab16.reshape(rows, 8, 2), jnp.int32)
    cpl = coords.reshape(N, 2)
    cx = cpl[:, 0]
    cy = cpl[:, 1]
    mesh = plsc.VectorSubcoreMesh(core_axis_name="c", subcore_axis_name="s")
    run = functools.partial(
        pl.kernel,
        mesh=mesh,
        compiler_params=pltpu.CompilerParams(
            needs_layout_passes=False, use_tc_tiling_on_sc=False),
        out_type=jax.ShapeDtypeStruct((N,), jnp.float32),
        scratch_types=[
            pltpu.VMEM((SUB,), jnp.float32),
            pltpu.VMEM((SUB,), jnp.float32),
            pltpu.VMEM((SUB,), jnp.int32),
            pltpu.VMEM((SUB,), jnp.int32),
            pltpu.VMEM((SUB, 8), jnp.int32),
            pltpu.VMEM((SUB, 8), jnp.int32),
            pltpu.VMEM((SUB,), jnp.float32),
            pltpu.VMEM_SHARED((SC_ROWS, 8), jnp.int32),
            pltpu.SemaphoreType.DMA,
        ],
    )(_sampler_body)
    out = run(cx, cy, tabw)
    return out.reshape(B, H, W, 1)
